# baseline (speedup 1.0000x reference)
import jax
import jax.numpy as jnp
from jax.experimental import pallas as pl
from jax.experimental.pallas import tpu as pltpu

N_NODES = 100000
IN_DIM = 128
NUM_CLASSES = 10
N_GRAPHS = 512
HP = 16

R = 2000
NBLK = N_NODES // R


def _body(x_ref, b3_ref, wt_ref, bias_ref, out_ref, acc_ref):
    i = pl.program_id(0)

    @pl.when(i == 0)
    def _():
        acc_ref[...] = jnp.zeros_like(acc_ref)

    h = jnp.dot(x_ref[...], wt_ref[...], preferred_element_type=jnp.float32)
    lane = jax.lax.broadcasted_iota(jnp.int32, (R, HP), 1)
    h_aug = jnp.where(lane == NUM_CLASSES, 1.0, h)

    bids = b3_ref[0, 0, :]
    seg = jax.lax.broadcasted_iota(jnp.int32, (N_GRAPHS, R), 0)
    onehot_t = (seg == bids[None, :]).astype(jnp.bfloat16)

    acc_ref[...] += jnp.dot(onehot_t, h_aug.astype(jnp.bfloat16),
                            preferred_element_type=jnp.float32)

    @pl.when(i == NBLK - 1)
    def _():
        a = acc_ref[...]
        out_ref[...] = a[:, :NUM_CLASSES] + a[:, NUM_CLASSES:NUM_CLASSES + 1] * bias_ref[...]


def kernel(x, edge_index, batch, W, b):
    del edge_index
    wt_pad = jnp.zeros((IN_DIM, HP), jnp.float32).at[:, :NUM_CLASSES].set(W.T)
    bias = b.reshape(1, NUM_CLASSES)
    batch3 = batch.reshape(NBLK, 1, R)

    out = pl.pallas_call(
        _body,
        grid=(NBLK,),
        in_specs=[
            pl.BlockSpec((R, IN_DIM), lambda i: (i, 0)),
            pl.BlockSpec((1, 1, R), lambda i: (i, 0, 0)),
            pl.BlockSpec((IN_DIM, HP), lambda i: (0, 0)),
            pl.BlockSpec((1, NUM_CLASSES), lambda i: (0, 0)),
        ],
        out_specs=pl.BlockSpec((N_GRAPHS, NUM_CLASSES), lambda i: (0, 0)),
        out_shape=jax.ShapeDtypeStruct((N_GRAPHS, NUM_CLASSES), jnp.float32),
        scratch_shapes=[pltpu.VMEM((N_GRAPHS, HP), jnp.float32)],
        compiler_params=pltpu.CompilerParams(
            dimension_semantics=("arbitrary",),
        ),
    )(x, batch3, wt_pad, bias)
    return out


# speedup vs baseline: 1.1980x; 1.1980x over previous
import jax
import jax.numpy as jnp
from jax.experimental import pallas as pl
from jax.experimental.pallas import tpu as pltpu

N_NODES = 100000
IN_DIM = 128
NUM_CLASSES = 10
N_GRAPHS = 512
HP = 16

R = 2000
NBLK = N_NODES // R
WIN = 64


def _body(x_ref, b3_ref, wt_ref, bias_ref, out_ref, acc_ref):
    i = pl.program_id(0)

    @pl.when(i == 0)
    def _():
        acc_ref[...] = jnp.zeros_like(acc_ref)

    h = jnp.dot(x_ref[...], wt_ref[...], preferred_element_type=jnp.float32)
    lane = jax.lax.broadcasted_iota(jnp.int32, (R, HP), 1)
    h_aug = jnp.where(lane == NUM_CLASSES, 1.0, h)

    bids = b3_ref[0, 0, :]
    h_bf = h_aug.astype(jnp.bfloat16)

    g0 = jnp.minimum((bids[0] // 8) * 8, N_GRAPHS - WIN)
    span_ok = (bids[R - 1] - g0) < WIN

    @pl.when(span_ok)
    def _():
        rel = bids - g0
        seg = jax.lax.broadcasted_iota(jnp.int32, (WIN, R), 0)
        onehot_t = (seg == rel[None, :]).astype(jnp.bfloat16)
        upd = jnp.dot(onehot_t, h_bf, preferred_element_type=jnp.float32)
        acc_ref[pl.ds(g0, WIN), :] += upd

    @pl.when(jnp.logical_not(span_ok))
    def _():
        seg = jax.lax.broadcasted_iota(jnp.int32, (N_GRAPHS, R), 0)
        onehot_t = (seg == bids[None, :]).astype(jnp.bfloat16)
        acc_ref[...] += jnp.dot(onehot_t, h_bf, preferred_element_type=jnp.float32)

    @pl.when(i == NBLK - 1)
    def _():
        a = acc_ref[...]
        out_ref[...] = a[:, :NUM_CLASSES] + a[:, NUM_CLASSES:NUM_CLASSES + 1] * bias_ref[...]


def kernel(x, edge_index, batch, W, b):
    del edge_index
    wt_pad = jnp.zeros((IN_DIM, HP), jnp.float32).at[:, :NUM_CLASSES].set(W.T)
    bias = b.reshape(1, NUM_CLASSES)
    batch3 = batch.reshape(NBLK, 1, R)

    out = pl.pallas_call(
        _body,
        grid=(NBLK,),
        in_specs=[
            pl.BlockSpec((R, IN_DIM), lambda i: (i, 0)),
            pl.BlockSpec((1, 1, R), lambda i: (i, 0, 0)),
            pl.BlockSpec((IN_DIM, HP), lambda i: (0, 0)),
            pl.BlockSpec((1, NUM_CLASSES), lambda i: (0, 0)),
        ],
        out_specs=pl.BlockSpec((N_GRAPHS, NUM_CLASSES), lambda i: (0, 0)),
        out_shape=jax.ShapeDtypeStruct((N_GRAPHS, NUM_CLASSES), jnp.float32),
        scratch_shapes=[pltpu.VMEM((N_GRAPHS, HP), jnp.float32)],
        compiler_params=pltpu.CompilerParams(
            dimension_semantics=("arbitrary",),
        ),
    )(x, batch3, wt_pad, bias)
    return out
